# baseline (speedup 1.0000x reference)
import jax
import jax.numpy as jnp
from jax.experimental import pallas as pl
from jax.experimental.pallas import tpu as pltpu

N_NODES_ = 10000
D_FEAT_ = 128
PE_DIM_ = 256
PE_EMBED_DIM_ = 512
BLK = 2000
GRID = N_NODES_ // BLK


def _manual_kernel(x_hbm, pe_hbm, w_ref, b_ref, out_hbm,
                   in_buf, out_buf, sem_in, sem_out, sem_x):
    x_copy = pltpu.make_async_copy(x_hbm, out_hbm.at[:, :D_FEAT_], sem_x)
    x_copy.start()

    def cp_in(i, slot):
        return pltpu.make_async_copy(
            pe_hbm.at[pl.ds(i * BLK, BLK), :], in_buf.at[slot], sem_in.at[slot])

    def cp_out(i, slot):
        return pltpu.make_async_copy(
            out_buf.at[slot], out_hbm.at[pl.ds(i * BLK, BLK), D_FEAT_:],
            sem_out.at[slot])

    cp_in(0, 0).start()
    for i in range(GRID):
        slot = i % 2
        if i + 1 < GRID:
            cp_in(i + 1, (i + 1) % 2).start()
        cp_in(i, slot).wait()
        if i >= 2:
            cp_out(i - 2, slot).wait()
        acc = jnp.dot(in_buf[slot], w_ref[:], preferred_element_type=jnp.float32)
        out_buf[slot] = acc + b_ref[:]
        cp_out(i, slot).start()
    cp_out(GRID - 2, GRID % 2).wait()
    cp_out(GRID - 1, (GRID - 1) % 2).wait()
    x_copy.wait()


def kernel(x, e_index, pos_enc, W, b):
    del e_index
    n = x.shape[0]
    out = pl.pallas_call(
        _manual_kernel,
        in_specs=[
            pl.BlockSpec(memory_space=pltpu.MemorySpace.HBM),
            pl.BlockSpec(memory_space=pltpu.MemorySpace.HBM),
            pl.BlockSpec(memory_space=pltpu.MemorySpace.VMEM),
            pl.BlockSpec(memory_space=pltpu.MemorySpace.VMEM),
        ],
        out_specs=pl.BlockSpec(memory_space=pltpu.MemorySpace.HBM),
        out_shape=jax.ShapeDtypeStruct((n, D_FEAT_ + PE_EMBED_DIM_), jnp.float32),
        scratch_shapes=[
            pltpu.VMEM((2, BLK, PE_DIM_), jnp.float32),
            pltpu.VMEM((2, BLK, PE_EMBED_DIM_), jnp.float32),
            pltpu.SemaphoreType.DMA((2,)),
            pltpu.SemaphoreType.DMA((2,)),
            pltpu.SemaphoreType.DMA,
        ],
    )(x, pos_enc, W, b)
    return out


# speedup vs baseline: 9.0720x; 9.0720x over previous
import jax
import jax.numpy as jnp
from jax.experimental import pallas as pl
from jax.experimental.pallas import tpu as pltpu

D_FEAT_ = 128
PE_DIM_ = 256
PE_EMBED_DIM_ = 512
BLOCK_M = 10000


def _fused_kernel(x_ref, pe_ref, w_ref, b_ref, out_ref):
    out_ref[:, :D_FEAT_] = x_ref[:]
    acc = jnp.dot(pe_ref[:], w_ref[:], preferred_element_type=jnp.float32)
    out_ref[:, D_FEAT_:] = acc + b_ref[:]


def kernel(x, e_index, pos_enc, W, b):
    del e_index
    n = x.shape[0]
    grid = (n // BLOCK_M,)
    out = pl.pallas_call(
        _fused_kernel,
        grid=grid,
        in_specs=[
            pl.BlockSpec((BLOCK_M, D_FEAT_), lambda i: (i, 0)),
            pl.BlockSpec((BLOCK_M, PE_DIM_), lambda i: (i, 0)),
            pl.BlockSpec((PE_DIM_, PE_EMBED_DIM_), lambda i: (0, 0)),
            pl.BlockSpec((PE_EMBED_DIM_,), lambda i: (0,)),
        ],
        out_specs=pl.BlockSpec((BLOCK_M, D_FEAT_ + PE_EMBED_DIM_), lambda i: (i, 0)),
        out_shape=jax.ShapeDtypeStruct((n, D_FEAT_ + PE_EMBED_DIM_), jnp.float32),
        compiler_params=pltpu.CompilerParams(
            dimension_semantics=("arbitrary",),
        ),
    )(x, pos_enc, W, b)
    return out


# speedup vs baseline: 10.6928x; 1.1787x over previous
import jax
import jax.numpy as jnp
from jax.experimental import pallas as pl
from jax.experimental.pallas import tpu as pltpu

N_NODES_ = 10000
D_FEAT_ = 128
PE_DIM_ = 256
PE_EMBED_DIM_ = 512
OUT_D_ = D_FEAT_ + PE_EMBED_DIM_

SIZES = (1000, 4000, 4000, 1000)
OFFS = (0, 1000, 5000, 9000)
G = len(SIZES)


def _manual_kernel(x_hbm, pe_hbm, w_ref, b_ref, out_hbm,
                   pe_buf, stage, sem_x, sem_pe, sem_out):
    def x_in(i):
        o, m = OFFS[i], SIZES[i]
        return pltpu.make_async_copy(
            x_hbm.at[pl.ds(o, m), :], stage.at[pl.ds(o, m), :D_FEAT_],
            sem_x.at[i])

    def pe_in(i):
        o, m = OFFS[i], SIZES[i]
        return pltpu.make_async_copy(
            pe_hbm.at[pl.ds(o, m), :], pe_buf.at[pl.ds(o, m), :],
            sem_pe.at[i])

    def out_cp(i):
        o, m = OFFS[i], SIZES[i]
        return pltpu.make_async_copy(
            stage.at[pl.ds(o, m), :], out_hbm.at[pl.ds(o, m), :],
            sem_out.at[i])

    for i in range(G):
        x_in(i).start()
        pe_in(i).start()
    for i in range(G):
        o, m = OFFS[i], SIZES[i]
        pe_in(i).wait()
        acc = jnp.dot(pe_buf[pl.ds(o, m), :], w_ref[:],
                      preferred_element_type=jnp.float32)
        stage[pl.ds(o, m), D_FEAT_:] = acc + b_ref[:]
        x_in(i).wait()
        out_cp(i).start()
    for i in range(G):
        out_cp(i).wait()


def kernel(x, e_index, pos_enc, W, b):
    del e_index
    n = x.shape[0]
    out = pl.pallas_call(
        _manual_kernel,
        in_specs=[
            pl.BlockSpec(memory_space=pltpu.MemorySpace.HBM),
            pl.BlockSpec(memory_space=pltpu.MemorySpace.HBM),
            pl.BlockSpec(memory_space=pltpu.MemorySpace.VMEM),
            pl.BlockSpec(memory_space=pltpu.MemorySpace.VMEM),
        ],
        out_specs=pl.BlockSpec(memory_space=pltpu.MemorySpace.HBM),
        out_shape=jax.ShapeDtypeStruct((n, OUT_D_), jnp.float32),
        scratch_shapes=[
            pltpu.VMEM((N_NODES_, PE_DIM_), jnp.float32),
            pltpu.VMEM((N_NODES_, OUT_D_), jnp.float32),
            pltpu.SemaphoreType.DMA((G,)),
            pltpu.SemaphoreType.DMA((G,)),
            pltpu.SemaphoreType.DMA((G,)),
        ],
    )(x, pos_enc, W, b)
    return out


# speedup vs baseline: 10.8384x; 1.0136x over previous
import jax
import jax.numpy as jnp
from jax.experimental import pallas as pl
from jax.experimental.pallas import tpu as pltpu

D_FEAT_ = 128
PE_DIM_ = 256
PE_EMBED_DIM_ = 512
BLOCK_M = 5000


def _fused_kernel(x_ref, pe_ref, w_ref, b_ref, out_ref):
    out_ref[:, :D_FEAT_] = x_ref[:]
    acc = jnp.dot(pe_ref[:], w_ref[:], preferred_element_type=jnp.float32,
                  precision=jax.lax.Precision.DEFAULT)
    out_ref[:, D_FEAT_:] = acc + b_ref[:]


def kernel(x, e_index, pos_enc, W, b):
    del e_index
    n = x.shape[0]
    grid = (n // BLOCK_M,)
    out = pl.pallas_call(
        _fused_kernel,
        grid=grid,
        in_specs=[
            pl.BlockSpec((BLOCK_M, D_FEAT_), lambda i: (i, 0)),
            pl.BlockSpec((BLOCK_M, PE_DIM_), lambda i: (i, 0)),
            pl.BlockSpec((PE_DIM_, PE_EMBED_DIM_), lambda i: (0, 0)),
            pl.BlockSpec((PE_EMBED_DIM_,), lambda i: (0,)),
        ],
        out_specs=pl.BlockSpec((BLOCK_M, D_FEAT_ + PE_EMBED_DIM_), lambda i: (i, 0)),
        out_shape=jax.ShapeDtypeStruct((n, D_FEAT_ + PE_EMBED_DIM_), jnp.float32),
        compiler_params=pltpu.CompilerParams(
            dimension_semantics=("arbitrary",),
        ),
    )(x, pos_enc, W, b)
    return out
